# Initial kernel scaffold; baseline (speedup 1.0000x reference)
import functools

import jax
import jax.numpy as jnp
from jax import lax
from jax.experimental import pallas as pl
from jax.experimental.pallas import tpu as pltpu
from jax.experimental.pallas import tpu_sc as plsc

N = 10000
NP = 10240
E = 320000
NC = 2
NS = 16
NW = NC * NS
C = 128
G = 80
EP = NW * G * C
RPT = NP // NS
D = 128
H = 128
OUT = 64
BR = 512
NB = NP // BR

_f32 = jnp.float32


def _deg_body(dst3, zeros16, ones_hbm, out, didx, cidx, ones_v, acc):
    c = lax.axis_index("c")
    s = lax.axis_index("s")
    wid = c * NS + s
    pltpu.sync_copy(dst3.at[wid], didx)
    pltpu.sync_copy(ones_hbm, ones_v)
    pltpu.sync_copy(zeros16.at[pl.ds(s * RPT, RPT)], acc.at[pl.ds(s * RPT, RPT)])
    plsc.subcore_barrier()

    def body(g, carry):
        pltpu.sync_copy(didx.at[g], cidx)
        pltpu.sync_copy(ones_v, acc.at[cidx], add=True)
        return carry

    lax.fori_loop(0, G, body, 0)
    plsc.subcore_barrier()
    pltpu.sync_copy(acc.at[pl.ds(s * RPT, RPT)],
                    out.at[c].at[pl.ds(s * RPT, RPT)])


def _agg_body(u, src3, dst3, zeros128, out,
              sidx, didx, cs, cd, gbuf, acc):
    c = lax.axis_index("c")
    s = lax.axis_index("s")
    wid = c * NS + s
    pltpu.sync_copy(src3.at[wid], sidx)
    pltpu.sync_copy(dst3.at[wid], didx)
    pltpu.sync_copy(zeros128.at[pl.ds(s * RPT, RPT)], acc.at[pl.ds(s * RPT, RPT)])
    plsc.subcore_barrier()

    def body(g, carry):
        pltpu.sync_copy(sidx.at[g], cs)
        pltpu.sync_copy(didx.at[g], cd)
        pltpu.sync_copy(u.at[cs], gbuf)
        pltpu.sync_copy(gbuf, acc.at[cd], add=True)
        return carry

    lax.fori_loop(0, G, body, 0)
    plsc.subcore_barrier()
    pltpu.sync_copy(acc.at[pl.ds(s * RPT, RPT)],
                    out.at[c].at[pl.ds(s * RPT, RPT)])


_sc_mesh = plsc.VectorSubcoreMesh(core_axis_name="c", subcore_axis_name="s")

_deg_kernel = pl.kernel(
    _deg_body,
    out_type=jax.ShapeDtypeStruct((NC, NP, 16), _f32),
    mesh=_sc_mesh,
    scratch_types=[
        pltpu.VMEM((G, C), jnp.int32),
        pltpu.VMEM((C,), jnp.int32),
        pltpu.VMEM((C, 16), _f32),
        pltpu.VMEM_SHARED((NP, 16), _f32),
    ],
)

_agg_kernel = pl.kernel(
    _agg_body,
    out_type=jax.ShapeDtypeStruct((NC, NP, D), _f32),
    mesh=_sc_mesh,
    scratch_types=[
        pltpu.VMEM((G, C), jnp.int32),
        pltpu.VMEM((G, C), jnp.int32),
        pltpu.VMEM((C,), jnp.int32),
        pltpu.VMEM((C,), jnp.int32),
        pltpu.VMEM((C, D), _f32),
        pltpu.VMEM_SHARED((NP, D), _f32),
    ],
)


def _dinv_body(d0, d1, o):
    i = pl.program_id(0)
    deg = d0[:, 0:1] + d1[:, 0:1] + 1.0
    r = lax.rsqrt(deg)
    rows = i * BR + lax.broadcasted_iota(jnp.int32, (BR, 1), 0)
    r = jnp.where(rows < N, r, 0.0)
    o[...] = jnp.broadcast_to(r, (BR, D))


def _emb_body(x, we, be, w1, dinv, o):
    h0 = jnp.dot(x[...], we[...], preferred_element_type=_f32) + be[...]
    o[...] = jnp.dot(h0, w1[...], preferred_element_type=_f32) * dinv[...]


def _mid_body(p0, p1, u, dinv, b, w, o):
    h = jnp.maximum(dinv[...] * (p0[...] + p1[...] + u[...]) + b[...], 0.0)
    o[...] = jnp.dot(h, w[...], preferred_element_type=_f32) * dinv[...]


def _fin_body(p0, p1, u, dinv, b, wf1, bf1, wf2, bf2, o, acc):
    i = pl.program_id(0)

    @pl.when(i == 0)
    def _():
        acc[...] = jnp.zeros_like(acc)

    h = jnp.maximum(dinv[...] * (p0[...] + p1[...] + u[...]) + b[...], 0.0)
    rows = i * BR + lax.broadcasted_iota(jnp.int32, (BR, 1), 0)
    h = jnp.where(rows < N, h, 0.0)
    acc[0:1, :] += jnp.sum(h, axis=0, keepdims=True)

    @pl.when(i == NB - 1)
    def _():
        g = acc[0:1, :] * (1.0 / N)
        z = jnp.maximum(jnp.dot(g, wf1[...], preferred_element_type=_f32)
                        + bf1[...], 0.0)
        o[...] = jnp.dot(z, wf2[...], preferred_element_type=_f32) + bf2[...]


def _rows_spec():
    return pl.BlockSpec((BR, D), lambda i: (i, 0))


def _full_spec(shape):
    return pl.BlockSpec(shape, lambda i: tuple(0 for _ in shape))


_dinv_kernel = pl.pallas_call(
    _dinv_body,
    grid=(NB,),
    in_specs=[pl.BlockSpec((BR, 16), lambda i: (i, 0)),
              pl.BlockSpec((BR, 16), lambda i: (i, 0))],
    out_specs=_rows_spec(),
    out_shape=jax.ShapeDtypeStruct((NP, D), _f32),
)

_emb_kernel = pl.pallas_call(
    _emb_body,
    grid=(NB,),
    in_specs=[_rows_spec(), _full_spec((D, H)), _full_spec((1, H)),
              _full_spec((H, H)), _rows_spec()],
    out_specs=_rows_spec(),
    out_shape=jax.ShapeDtypeStruct((NP, H), _f32),
)

_mid_kernel = pl.pallas_call(
    _mid_body,
    grid=(NB,),
    in_specs=[_rows_spec(), _rows_spec(), _rows_spec(), _rows_spec(),
              _full_spec((1, H)), _full_spec((H, H))],
    out_specs=_rows_spec(),
    out_shape=jax.ShapeDtypeStruct((NP, H), _f32),
)

_fin_kernel = pl.pallas_call(
    _fin_body,
    grid=(NB,),
    in_specs=[_rows_spec(), _rows_spec(), _rows_spec(), _rows_spec(),
              _full_spec((1, H)), _full_spec((H, H)), _full_spec((1, H)),
              _full_spec((H, OUT)), _full_spec((1, OUT))],
    out_specs=pl.BlockSpec((1, OUT), lambda i: (0, 0)),
    out_shape=jax.ShapeDtypeStruct((1, OUT), _f32),
    scratch_shapes=[pltpu.VMEM((8, H), _f32)],
    compiler_params=pltpu.CompilerParams(dimension_semantics=("arbitrary",)),
)


def kernel(x, edge_index, W_emb, b_emb, W1, b1, W2, b2, W3, b3,
           W_fc1, b_fc1, W_fc2, b_fc2):
    ei = edge_index.astype(jnp.int32)
    src3 = jnp.pad(ei[0], (0, EP - E), constant_values=N).reshape(NW, G, C)
    dst3 = jnp.pad(ei[1], (0, EP - E), constant_values=N).reshape(NW, G, C)

    zeros16 = jnp.zeros((NP, 16), _f32)
    zeros128 = jnp.zeros((NP, D), _f32)
    ones16 = jnp.ones((C, 16), _f32)

    degP = _deg_kernel(dst3, zeros16, ones16)
    dinv = _dinv_kernel(degP[0], degP[1])

    xp = jnp.pad(x, ((0, NP - N), (0, 0)))
    b_emb2 = b_emb.reshape(1, H)

    u = _emb_kernel(xp, W_emb, b_emb2, W1, dinv)
    for W_next, b_prev in ((W2, b1), (W3, b2)):
        aggP = _agg_kernel(u, src3, dst3, zeros128)
        u = _mid_kernel(aggP[0], aggP[1], u, dinv, b_prev.reshape(1, H), W_next)

    aggP = _agg_kernel(u, src3, dst3, zeros128)
    out = _fin_kernel(aggP[0], aggP[1], u, dinv, b3.reshape(1, H),
                      W_fc1, b_fc1.reshape(1, H), W_fc2, b_fc2.reshape(1, OUT))
    return out


# speedup vs baseline: 4.7309x; 4.7309x over previous
import functools

import jax
import jax.numpy as jnp
from jax import lax
from jax.experimental import pallas as pl
from jax.experimental.pallas import tpu as pltpu
from jax.experimental.pallas import tpu_sc as plsc

N = 10000
NP = 10240
E = 320000
NC = 2
NS = 16
NW = NC * NS
C = 128
G = 80
EP = NW * G * C
RPT = NP // NS
D = 128
H = 128
OUT = 64
BR = 512
NB = NP // BR

_f32 = jnp.float32


def _agg_body(u, src3, dst3, zeros128, out,
              sidx, didx, gbuf, acc):
    c = lax.axis_index("c")
    s = lax.axis_index("s")
    wid = c * NS + s
    pltpu.sync_copy(src3.at[wid], sidx)
    pltpu.sync_copy(dst3.at[wid], didx)
    pltpu.sync_copy(zeros128.at[pl.ds(s * RPT, RPT)], acc.at[pl.ds(s * RPT, RPT)])
    plsc.subcore_barrier()

    def body(g, carry):
        pltpu.sync_copy(u.at[sidx.at[g]], gbuf)
        pltpu.sync_copy(gbuf, acc.at[didx.at[g]], add=True)
        return carry

    lax.fori_loop(0, G, body, 0)
    plsc.subcore_barrier()
    pltpu.sync_copy(acc.at[pl.ds(s * RPT, RPT)],
                    out.at[c].at[pl.ds(s * RPT, RPT)])


_sc_mesh = plsc.VectorSubcoreMesh(core_axis_name="c", subcore_axis_name="s")

_agg_kernel = pl.kernel(
    _agg_body,
    out_type=jax.ShapeDtypeStruct((NC, NP, D), _f32),
    mesh=_sc_mesh,
    scratch_types=[
        pltpu.VMEM((G, C), jnp.int32),
        pltpu.VMEM((G, C), jnp.int32),
        pltpu.VMEM((C, D), _f32),
        pltpu.VMEM_SHARED((NP, D), _f32),
    ],
)


def _dinv_body(d0, d1, o):
    i = pl.program_id(0)
    deg = d0[:, 0:1] + d1[:, 0:1] + 1.0
    r = lax.rsqrt(deg)
    rows = i * BR + lax.broadcasted_iota(jnp.int32, (BR, 1), 0)
    r = jnp.where(rows < N, r, 0.0)
    o[...] = jnp.broadcast_to(r, (BR, D))


def _emb_body(x, we, be, w1, dinv, o):
    h0 = jnp.dot(x[...], we[...], preferred_element_type=_f32) + be[...]
    o[...] = jnp.dot(h0, w1[...], preferred_element_type=_f32) * dinv[...]


def _mid_body(p0, p1, u, dinv, b, w, o):
    h = jnp.maximum(dinv[...] * (p0[...] + p1[...] + u[...]) + b[...], 0.0)
    o[...] = jnp.dot(h, w[...], preferred_element_type=_f32) * dinv[...]


def _fin_body(p0, p1, u, dinv, b, wf1, bf1, wf2, bf2, o, acc):
    i = pl.program_id(0)

    @pl.when(i == 0)
    def _():
        acc[...] = jnp.zeros_like(acc)

    h = jnp.maximum(dinv[...] * (p0[...] + p1[...] + u[...]) + b[...], 0.0)
    rows = i * BR + lax.broadcasted_iota(jnp.int32, (BR, 1), 0)
    h = jnp.where(rows < N, h, 0.0)
    acc[0:1, :] += jnp.sum(h, axis=0, keepdims=True)

    @pl.when(i == NB - 1)
    def _():
        g = acc[0:1, :] * (1.0 / N)
        z = jnp.maximum(jnp.dot(g, wf1[...], preferred_element_type=_f32)
                        + bf1[...], 0.0)
        o[...] = jnp.dot(z, wf2[...], preferred_element_type=_f32) + bf2[...]


def _rows_spec():
    return pl.BlockSpec((BR, D), lambda i: (i, 0))


def _full_spec(shape):
    return pl.BlockSpec(shape, lambda i: tuple(0 for _ in shape))


_dinv_kernel = pl.pallas_call(
    _dinv_body,
    grid=(NB,),
    in_specs=[_rows_spec(), _rows_spec()],
    out_specs=_rows_spec(),
    out_shape=jax.ShapeDtypeStruct((NP, D), _f32),
)

_emb_kernel = pl.pallas_call(
    _emb_body,
    grid=(NB,),
    in_specs=[_rows_spec(), _full_spec((D, H)), _full_spec((1, H)),
              _full_spec((H, H)), _rows_spec()],
    out_specs=_rows_spec(),
    out_shape=jax.ShapeDtypeStruct((NP, H), _f32),
)

_mid_kernel = pl.pallas_call(
    _mid_body,
    grid=(NB,),
    in_specs=[_rows_spec(), _rows_spec(), _rows_spec(), _rows_spec(),
              _full_spec((1, H)), _full_spec((H, H))],
    out_specs=_rows_spec(),
    out_shape=jax.ShapeDtypeStruct((NP, H), _f32),
)

_fin_kernel = pl.pallas_call(
    _fin_body,
    grid=(NB,),
    in_specs=[_rows_spec(), _rows_spec(), _rows_spec(), _rows_spec(),
              _full_spec((1, H)), _full_spec((H, H)), _full_spec((1, H)),
              _full_spec((H, OUT)), _full_spec((1, OUT))],
    out_specs=pl.BlockSpec((1, OUT), lambda i: (0, 0)),
    out_shape=jax.ShapeDtypeStruct((1, OUT), _f32),
    scratch_shapes=[pltpu.VMEM((8, H), _f32)],
    compiler_params=pltpu.CompilerParams(dimension_semantics=("arbitrary",)),
)


def kernel(x, edge_index, W_emb, b_emb, W1, b1, W2, b2, W3, b3,
           W_fc1, b_fc1, W_fc2, b_fc2):
    ei = edge_index.astype(jnp.int32)
    src3 = jnp.pad(ei[0], (0, EP - E), constant_values=N).reshape(NW, G, C)
    dst3 = jnp.pad(ei[1], (0, EP - E), constant_values=N).reshape(NW, G, C)

    zeros128 = jnp.zeros((NP, D), _f32)
    ones128 = jnp.ones((NP, D), _f32)

    degP = _agg_kernel(ones128, src3, dst3, zeros128)
    dinv = _dinv_kernel(degP[0], degP[1])

    xp = jnp.pad(x, ((0, NP - N), (0, 0)))
    b_emb2 = b_emb.reshape(1, H)

    u = _emb_kernel(xp, W_emb, b_emb2, W1, dinv)
    for W_next, b_prev in ((W2, b1), (W3, b2)):
        aggP = _agg_kernel(u, src3, dst3, zeros128)
        u = _mid_kernel(aggP[0], aggP[1], u, dinv, b_prev.reshape(1, H), W_next)

    aggP = _agg_kernel(u, src3, dst3, zeros128)
    out = _fin_kernel(aggP[0], aggP[1], u, dinv, b3.reshape(1, H),
                      W_fc1, b_fc1.reshape(1, H), W_fc2, b_fc2.reshape(1, OUT))
    return out
